# baseline (device time: 25574 ns/iter reference)
import jax
import jax.numpy as jnp
from jax import lax
from jax.experimental import pallas as pl
from jax.experimental.pallas import tpu as pltpu

Y_DIM = 4


def kernel(x, assign, W1, W2):
    t, d = x.shape
    e_per, _, f = W1.shape
    ts = t // Y_DIM
    assign2 = assign.reshape(t, 1)

    def body(x_ref, a_ref, w1_ref, w2_ref, out_ref,
             xs_ref, xr_ref, ar_ref, rr_ref, b_ref, g_ref,
             w1c_ref, w2c_ref, send_sems, recv_sems):
        my_x = lax.axis_index("x")
        my_y = lax.axis_index("y")
        my_z = lax.axis_index("z")
        partner = (1 - my_x, my_y, my_z)
        r0 = ts * my_y

        barrier_sem = pltpu.get_barrier_semaphore()
        pl.semaphore_signal(barrier_sem, inc=1, device_id=partner,
                            device_id_type=pl.DeviceIdType.MESH)
        for j in range(1, Y_DIM):
            peer = (my_x, (my_y + j) % Y_DIM, my_z)
            pl.semaphore_signal(barrier_sem, inc=1, device_id=peer,
                                device_id_type=pl.DeviceIdType.MESH)
        pl.semaphore_wait(barrier_sem, Y_DIM)

        xs_ref[:, :] = x_ref[pl.ds(r0, ts), :].astype(jnp.bfloat16)
        rdma_x = pltpu.make_async_remote_copy(
            src_ref=xs_ref, dst_ref=xr_ref,
            send_sem=send_sems.at[0], recv_sem=recv_sems.at[0],
            device_id=partner, device_id_type=pl.DeviceIdType.MESH)
        rdma_a = pltpu.make_async_remote_copy(
            src_ref=a_ref.at[pl.ds(r0, ts)], dst_ref=ar_ref,
            send_sem=send_sems.at[1], recv_sem=recv_sems.at[1],
            device_id=partner, device_id_type=pl.DeviceIdType.MESH)
        rdma_x.start()
        rdma_a.start()

        def my_experts_contrib(xs, am):
            acc = jnp.zeros((ts, d), jnp.float32)
            for e_loc in range(e_per):
                ge = e_loc + e_per * my_x
                h = jnp.maximum(
                    jnp.dot(xs, w1c_ref[e_loc],
                            preferred_element_type=jnp.float32), 0.0)
                y = jnp.dot(h.astype(jnp.bfloat16), w2c_ref[e_loc],
                            preferred_element_type=jnp.float32)
                acc = acc + (am == ge).astype(jnp.float32) * y
            return acc

        am_own = a_ref[pl.ds(r0, ts), :]
        a_contrib = jnp.zeros((ts, d), jnp.float32)
        for e_loc in range(e_per):
            w1c_ref[e_loc, :, :] = w1_ref[e_loc, :, :].astype(jnp.bfloat16)
            w2c_ref[e_loc, :, :] = w2_ref[e_loc, :, :].astype(jnp.bfloat16)
            ge = e_loc + e_per * my_x
            h = jnp.maximum(
                jnp.dot(xs_ref[:, :], w1c_ref[e_loc],
                        preferred_element_type=jnp.float32), 0.0)
            y = jnp.dot(h.astype(jnp.bfloat16), w2c_ref[e_loc],
                        preferred_element_type=jnp.float32)
            a_contrib = a_contrib + (am_own == ge).astype(jnp.float32) * y

        rdma_x.wait_recv()
        rdma_a.wait_recv()

        b_ref[:, :] = my_experts_contrib(
            xr_ref[:, :], ar_ref[:, :]).astype(jnp.bfloat16)
        rdma_b = pltpu.make_async_remote_copy(
            src_ref=b_ref, dst_ref=rr_ref,
            send_sem=send_sems.at[2], recv_sem=recv_sems.at[2],
            device_id=partner, device_id_type=pl.DeviceIdType.MESH)
        rdma_b.start()
        rdma_b.wait_recv()

        chunk = a_contrib + rr_ref[:, :].astype(jnp.float32)
        out_ref[pl.ds(r0, ts), :] = chunk
        g_ref[pl.ds(r0, ts), :] = chunk.astype(jnp.bfloat16)

        sends = []
        for j in range(1, Y_DIM):
            peer = (my_x, (my_y + j) % Y_DIM, my_z)
            s = pltpu.make_async_remote_copy(
                src_ref=g_ref.at[pl.ds(r0, ts)],
                dst_ref=g_ref.at[pl.ds(r0, ts)],
                send_sem=send_sems.at[2 + j], recv_sem=recv_sems.at[2 + j],
                device_id=peer, device_id_type=pl.DeviceIdType.MESH)
            s.start()
            sends.append(s)

        for j in range(1, Y_DIM):
            src_y = (my_y - j) % Y_DIM
            rs = ts * src_y
            recv = pltpu.make_async_remote_copy(
                src_ref=g_ref.at[pl.ds(rs, ts)],
                dst_ref=g_ref.at[pl.ds(rs, ts)],
                send_sem=send_sems.at[2 + j], recv_sem=recv_sems.at[2 + j],
                device_id=partner, device_id_type=pl.DeviceIdType.MESH)
            recv.wait_recv()
            out_ref[pl.ds(rs, ts), :] = g_ref[pl.ds(rs, ts), :].astype(
                jnp.float32)

        for r in (rdma_x, rdma_a, rdma_b, *sends):
            r.wait_send()

    return pl.pallas_call(
        body,
        out_shape=jax.ShapeDtypeStruct((t, d), jnp.float32),
        in_specs=[pl.BlockSpec(memory_space=pltpu.VMEM)] * 4,
        out_specs=pl.BlockSpec(memory_space=pltpu.VMEM),
        scratch_shapes=[
            pltpu.VMEM((ts, d), jnp.bfloat16),
            pltpu.VMEM((ts, d), jnp.bfloat16),
            pltpu.VMEM((ts, 1), jnp.int32),
            pltpu.VMEM((ts, d), jnp.bfloat16),
            pltpu.VMEM((ts, d), jnp.bfloat16),
            pltpu.VMEM((t, d), jnp.bfloat16),
            pltpu.VMEM((e_per, d, f), jnp.bfloat16),
            pltpu.VMEM((e_per, f, d), jnp.bfloat16),
            pltpu.SemaphoreType.DMA((6,)),
            pltpu.SemaphoreType.DMA((6,)),
        ],
        compiler_params=pltpu.CompilerParams(collective_id=0),
    )(x, assign2, W1, W2)


# device time: 24297 ns/iter; 1.0526x vs baseline; 1.0526x over previous
import jax
import jax.numpy as jnp
from jax import lax
from jax.experimental import pallas as pl
from jax.experimental.pallas import tpu as pltpu

Y_DIM = 4


def kernel(x, assign, W1, W2):
    t, d = x.shape
    e_per = W1.shape[0]
    ts = t // Y_DIM
    assign2 = assign.reshape(t, 1)
    x16 = x.astype(jnp.bfloat16)
    W1_16 = W1.astype(jnp.bfloat16)
    W2_16 = W2.astype(jnp.bfloat16)

    def body(x_ref, a_ref, w1_ref, w2_ref, out_ref,
             xr_ref, ar_ref, b_ref, gb_ref, send_sems, recv_sems):
        my_x = lax.axis_index("x")
        my_y = lax.axis_index("y")
        my_z = lax.axis_index("z")
        partner = (1 - my_x, my_y, my_z)
        r0 = ts * my_y

        barrier_sem = pltpu.get_barrier_semaphore()
        for j in range(Y_DIM):
            tgt = (1 - my_x, (my_y + j) % Y_DIM, my_z)
            pl.semaphore_signal(barrier_sem, inc=1, device_id=tgt,
                                device_id_type=pl.DeviceIdType.MESH)
        pl.semaphore_wait(barrier_sem, Y_DIM)

        rdma_x = pltpu.make_async_remote_copy(
            src_ref=x_ref.at[pl.ds(r0, ts)], dst_ref=xr_ref,
            send_sem=send_sems.at[0], recv_sem=recv_sems.at[0],
            device_id=partner, device_id_type=pl.DeviceIdType.MESH)
        rdma_a = pltpu.make_async_remote_copy(
            src_ref=a_ref.at[pl.ds(r0, ts)], dst_ref=ar_ref,
            send_sem=send_sems.at[1], recv_sem=recv_sems.at[1],
            device_id=partner, device_id_type=pl.DeviceIdType.MESH)
        rdma_x.start()
        rdma_a.start()

        def my_experts_contrib(xs, am, rows):
            acc = jnp.zeros((rows, d), jnp.float32)
            for e_loc in range(e_per):
                ge = e_loc + e_per * my_x
                h = jnp.maximum(
                    jnp.dot(xs, w1_ref[e_loc],
                            preferred_element_type=jnp.float32), 0.0)
                y = jnp.dot(h.astype(jnp.bfloat16), w2_ref[e_loc],
                            preferred_element_type=jnp.float32)
                acc = acc + (am == ge).astype(jnp.float32) * y
            return acc

        rdma_x.wait_recv()
        rdma_a.wait_recv()

        b_ref[:, :] = my_experts_contrib(
            xr_ref[:, :], ar_ref[:, :], ts).astype(jnp.bfloat16)
        sends = [rdma_x, rdma_a]
        for j in range(Y_DIM):
            tgt = (1 - my_x, (my_y + j) % Y_DIM, my_z)
            s = pltpu.make_async_remote_copy(
                src_ref=b_ref,
                dst_ref=gb_ref.at[pl.ds(r0, ts)],
                send_sem=send_sems.at[2 + j], recv_sem=recv_sems.at[2 + j],
                device_id=tgt, device_id_type=pl.DeviceIdType.MESH)
            s.start()
            sends.append(s)

        out_ref[:, :] = my_experts_contrib(x_ref[:, :], a_ref[:, :], t)

        for j in range(Y_DIM):
            rs = ts * ((my_y - j) % Y_DIM)
            recv = pltpu.make_async_remote_copy(
                src_ref=b_ref,
                dst_ref=gb_ref.at[pl.ds(rs, ts)],
                send_sem=send_sems.at[2 + j], recv_sem=recv_sems.at[2 + j],
                device_id=partner, device_id_type=pl.DeviceIdType.MESH)
            recv.wait_recv()
            out_ref[pl.ds(rs, ts), :] = (
                out_ref[pl.ds(rs, ts), :]
                + gb_ref[pl.ds(rs, ts), :].astype(jnp.float32))

        for r in sends:
            r.wait_send()

    return pl.pallas_call(
        body,
        out_shape=jax.ShapeDtypeStruct((t, d), jnp.float32),
        in_specs=[pl.BlockSpec(memory_space=pltpu.VMEM)] * 4,
        out_specs=pl.BlockSpec(memory_space=pltpu.VMEM),
        scratch_shapes=[
            pltpu.VMEM((ts, d), jnp.bfloat16),
            pltpu.VMEM((ts, 1), jnp.int32),
            pltpu.VMEM((ts, d), jnp.bfloat16),
            pltpu.VMEM((t, d), jnp.bfloat16),
            pltpu.SemaphoreType.DMA((6,)),
            pltpu.SemaphoreType.DMA((6,)),
        ],
        compiler_params=pltpu.CompilerParams(collective_id=0),
    )(x16, assign2, W1_16, W2_16)
